# Initial kernel scaffold; baseline (speedup 1.0000x reference)
import jax
import jax.numpy as jnp
from jax.experimental import pallas as pl


def kernel(w, x, src):
    raise NotImplementedError("write your pallas kernel here")


# speedup vs baseline: 3.7777x; 3.7777x over previous
import jax
import jax.numpy as jnp
from jax import lax
from jax.experimental import pallas as pl
from jax.experimental.pallas import tpu as pltpu
from jax.experimental.pallas import tpu_sc as plsc

_B, _N, _M, _K = 4096, 100, 128, 64
_R = _B * _N
_NC, _NS = 2, 16
_NW = _NC * _NS
_RPW = _R // _NW
_CHUNK = 128
_NCHUNK = _RPW // _CHUNK


def _body(w_hbm, k_hbm, s_hbm, out_hbm, w_v, k_v, s_v):
    cid = lax.axis_index("c")
    sid = lax.axis_index("s")
    wid = sid * _NC + cid
    row0 = wid * _RPW

    def chunk_body(g, carry):
        rbase = row0 + g * _CHUNK
        pltpu.sync_copy(w_hbm.at[pl.ds(rbase * _M, _CHUNK * _M)], w_v)
        pltpu.sync_copy(k_hbm.at[pl.ds(rbase * _K, _CHUNK * _K)], k_v)
        pltpu.sync_copy(s_hbm.at[pl.ds(rbase * _K, _CHUNK * _K)], s_v)
        base = rbase * _M

        def row_body(r, rcarry):
            for j in range(_K // 16):
                off = r * _K + j * 16
                idx = k_v[pl.ds(off, 16)] - base
                val = s_v[pl.ds(off, 16)]
                _, last = plsc.scan_count(idx)
                plsc.store_scatter(w_v, [idx], val, mask=last)
            return rcarry

        lax.fori_loop(0, _CHUNK, row_body, 0)
        pltpu.sync_copy(w_v, out_hbm.at[pl.ds(rbase * _M, _CHUNK * _M)])
        return carry

    lax.fori_loop(0, _NCHUNK, chunk_body, 0)


@jax.jit
def kernel(w, x, src):
    bi = lax.broadcasted_iota(jnp.int32, (_B, _N, _K), 0)
    ni = lax.broadcasted_iota(jnp.int32, (_B, _N, _K), 1)
    keys = (bi * _N + ni) * _M + x
    skey, ssrc = lax.sort(
        (keys.reshape(_R * _K), src.reshape(_R * _K)),
        dimension=0, is_stable=False, num_keys=1,
    )
    mesh = plsc.VectorSubcoreMesh(core_axis_name="c", subcore_axis_name="s")
    kfn = pl.kernel(
        _body,
        out_type=jax.ShapeDtypeStruct((_R * _M,), jnp.float32),
        mesh=mesh,
        compiler_params=pltpu.CompilerParams(needs_layout_passes=False),
        scratch_types=[
            pltpu.VMEM((_CHUNK * _M,), jnp.float32),
            pltpu.VMEM((_CHUNK * _K,), jnp.int32),
            pltpu.VMEM((_CHUNK * _K,), jnp.float32),
        ],
    )
    out = kfn(w.reshape(_R * _M), skey, ssrc)
    return out.reshape(_B, _N, _M)
